# baseline (speedup 1.0000x reference)
import jax
import jax.numpy as jnp
from jax import lax
from jax.experimental import pallas as pl
from jax.experimental.pallas import tpu as pltpu
from jax.experimental.pallas import tpu_sc as plsc

HIDDEN = 64
SCALE = 8.0

_NC = 2
_NS = 16
_NW = _NC * _NS

_C = 128
_B = 4096 * 200
_BPW = _B // _NW
_NCH = _BPW // _C
_D = 4


def _expand_mask(idx_v, rows_g, rows_p):
    def group_body(g, carry):
        iv = idx_v[pl.ds(g * 16, 16)]
        sv = jnp.where(iv != 0, jnp.float32(SCALE), jnp.float32(0.0))
        for sub in range(16):
            bc = sv[jnp.full((16,), sub, jnp.int32)]
            r = g * 16 + sub
            for q in range(4):
                sl = pl.ds(q * 16, 16)
                rows_p[r, sl] = rows_g[r, sl] * bc
        return carry

    lax.fori_loop(0, _C // 16, group_body, 0)


def _body(x_hbm, table_hbm, out_hbm, *scratch):
    idxs = scratch[0:_D]
    rowsg = scratch[_D:2 * _D]
    rowsp = scratch[2 * _D:3 * _D]
    semg = scratch[3 * _D:4 * _D]
    semo = scratch[4 * _D:5 * _D]

    wid = lax.axis_index("s") * _NC + lax.axis_index("c")
    base = wid * _BPW

    def out_slice(c):
        return out_hbm.at[pl.ds(base + c * _C, _C)]

    def start(c, b):
        pltpu.sync_copy(x_hbm.at[wid, c], idxs[b])
        pltpu.async_copy(table_hbm.at[idxs[b]], rowsg[b], semg[b])

    def quad_body(cc, carry):
        c0 = cc * 4
        for j in range(4):
            c = c0 + j
            b = j
            bn = (j + 3) % 4

            @pl.when(c + 3 < _NCH)
            def _prefetch(c=c, bn=bn):
                start(c + 3, bn)

            pltpu.make_async_copy(
                table_hbm.at[idxs[b]], rowsg[b], semg[b]
            ).wait()

            @pl.when(c >= 4)
            def _(c=c, b=b):
                pltpu.make_async_copy(
                    rowsp[b], out_slice(c - 4), semo[b]
                ).wait()

            _expand_mask(idxs[b], rowsg[b], rowsp[b])
            pltpu.async_copy(rowsp[b], out_slice(c), semo[b])
        return carry

    for c in range(3):
        start(c, c)

    lax.fori_loop(0, _NCH // 4, quad_body, 0)

    for j in range(4):
        c = _NCH - 4 + j
        pltpu.make_async_copy(rowsp[c % 4], out_slice(c), semo[c % 4]).wait()


def kernel(x, shared_weights):
    b_total = x.size
    assert b_total == _B

    xr = x.astype(jnp.int32).reshape(_NW, _NCH, _C)

    mesh = plsc.VectorSubcoreMesh(core_axis_name="c", subcore_axis_name="s")
    run = pl.kernel(
        _body,
        out_type=jax.ShapeDtypeStruct((_B, 128), jnp.float32),
        mesh=mesh,
        scratch_types=(
            [pltpu.VMEM((_C,), jnp.int32) for _ in range(_D)]
            + [pltpu.VMEM((_C, HIDDEN), jnp.float32) for _ in range(_D)]
            + [pltpu.VMEM((_C, 128), jnp.float32) for _ in range(_D)]
            + [pltpu.SemaphoreType.DMA for _ in range(_D)]
            + [pltpu.SemaphoreType.DMA for _ in range(_D)]
        ),
        compiler_params=pltpu.CompilerParams(use_tc_tiling_on_sc=False),
    )
    out = run(xr, shared_weights)
    return out.reshape(4096, 200, 128)[:, :, :HIDDEN]


# speedup vs baseline: 1.1664x; 1.1664x over previous
import jax
import jax.numpy as jnp
from jax import lax
from jax.experimental import pallas as pl
from jax.experimental.pallas import tpu as pltpu
from jax.experimental.pallas import tpu_sc as plsc

HIDDEN = 64
SCALE = 8.0

_NC = 2
_NS = 16
_NW = _NC * _NS

_K = 2
_C = _K * 128
_B = 4096 * 200
_BPW = _B // _NW
_NCH = _BPW // _C


def _fire_gathers(table_hbm, idx_v, rows_v, sem):
    for j in range(_K):
        pltpu.async_copy(
            table_hbm.at[idx_v.at[j]],
            rows_v.at[pl.ds(j * 128, 128)],
            sem,
        )


def _drain_gathers(table_hbm, idx_v, rows_v, sem):
    for j in range(_K):
        pltpu.make_async_copy(
            table_hbm.at[idx_v.at[j]],
            rows_v.at[pl.ds(j * 128, 128)],
            sem,
        ).wait()


def _body(x_hbm, table_hbm, out_hbm,
          idx0, idx1, rows0, rows1, semg0, semg1, semo0, semo1):
    wid = lax.axis_index("s") * _NC + lax.axis_index("c")
    base = wid * _BPW

    bufs = ((idx0, rows0, semg0, semo0), (idx1, rows1, semg1, semo1))

    def out_slice(c):
        return out_hbm.at[pl.ds(base + c * _C, _C)]

    def step(c, cur, other):
        idx_c, rows_c, semg_c, semo_c = cur
        idx_o, rows_o, semg_o, semo_o = other

        @pl.when(c + 1 < _NCH)
        def _prefetch():
            @pl.when(c >= 1)
            def _():
                pltpu.make_async_copy(rows_o, out_slice(c - 1), semo_o).wait()

            pltpu.sync_copy(x_hbm.at[wid, c + 1], idx_o)
            _fire_gathers(table_hbm, idx_o, rows_o, semg_o)

        _drain_gathers(table_hbm, idx_c, rows_c, semg_c)
        pltpu.async_copy(rows_c, out_slice(c), semo_c)

    def pair_body(cc, carry):
        c0 = cc * 2
        step(c0, bufs[0], bufs[1])
        step(c0 + 1, bufs[1], bufs[0])
        return carry

    pltpu.sync_copy(x_hbm.at[wid, 0], idx0)
    _fire_gathers(table_hbm, idx0, rows0, semg0)

    lax.fori_loop(0, _NCH // 2, pair_body, 0)

    pltpu.make_async_copy(rows0, out_slice(_NCH - 2), semo0).wait()
    pltpu.make_async_copy(rows1, out_slice(_NCH - 1), semo1).wait()


def kernel(x, shared_weights):
    b_total = x.size
    assert b_total == _B

    xr = x.astype(jnp.int32).reshape(_NW, _NCH, _K, 128)

    wpad = jnp.pad(shared_weights, ((0, 0), (0, 128 - HIDDEN)))
    row_ids = lax.broadcasted_iota(jnp.int32, wpad.shape, 0)
    wprep = jnp.where(row_ids == 0, jnp.float32(0.0),
                      wpad * jnp.float32(SCALE))

    mesh = plsc.VectorSubcoreMesh(core_axis_name="c", subcore_axis_name="s")
    run = pl.kernel(
        _body,
        out_type=jax.ShapeDtypeStruct((_B, 128), jnp.float32),
        mesh=mesh,
        scratch_types=[
            pltpu.VMEM((_K, 128), jnp.int32),
            pltpu.VMEM((_K, 128), jnp.int32),
            pltpu.VMEM((_C, 128), jnp.float32),
            pltpu.VMEM((_C, 128), jnp.float32),
            pltpu.SemaphoreType.DMA,
            pltpu.SemaphoreType.DMA,
            pltpu.SemaphoreType.DMA,
            pltpu.SemaphoreType.DMA,
        ],
        compiler_params=pltpu.CompilerParams(use_tc_tiling_on_sc=False),
    )
    out = run(xr, wprep)
    return out.reshape(4096, 200, 128)[:, :, :HIDDEN]


# speedup vs baseline: 1.2041x; 1.0323x over previous
import jax
import jax.numpy as jnp
from jax import lax
from jax.experimental import pallas as pl
from jax.experimental.pallas import tpu as pltpu
from jax.experimental.pallas import tpu_sc as plsc

HIDDEN = 64
SCALE = 8.0

_NC = 2
_NS = 16
_NW = _NC * _NS

_K = 2
_C = _K * 128
_B = 4096 * 200
_BPW = _B // _NW
_NCH = _BPW // _C


def _fire_gathers(table_hbm, idx_v, rows_v, sem):
    for j in range(_K):
        pltpu.async_copy(
            table_hbm.at[idx_v.at[j]],
            rows_v.at[pl.ds(j * 128, 128)],
            sem,
        )


def _drain_gathers(table_hbm, idx_v, rows_v, sem):
    for j in range(_K):
        pltpu.make_async_copy(
            table_hbm.at[idx_v.at[j]],
            rows_v.at[pl.ds(j * 128, 128)],
            sem,
        ).wait()


def _body(x_hbm, table_hbm, out_hbm,
          idx0, idx1, rows0, rows1, semg0, semg1, semo0, semo1):
    wid = lax.axis_index("s") * _NC + lax.axis_index("c")
    base = wid * _BPW

    bufs = ((idx0, rows0, semg0, semo0), (idx1, rows1, semg1, semo1))

    def out_slice(c):
        return out_hbm.at[pl.ds(base + c * _C, _C), pl.ds(0, HIDDEN)]

    def step(c, cur, other):
        idx_c, rows_c, semg_c, semo_c = cur
        idx_o, rows_o, semg_o, semo_o = other

        @pl.when(c + 1 < _NCH)
        def _prefetch():
            @pl.when(c >= 1)
            def _():
                pltpu.make_async_copy(
                    rows_o.at[:, pl.ds(0, HIDDEN)], out_slice(c - 1), semo_o
                ).wait()

            pltpu.sync_copy(x_hbm.at[wid, c + 1], idx_o)
            _fire_gathers(table_hbm, idx_o, rows_o, semg_o)

        _drain_gathers(table_hbm, idx_c, rows_c, semg_c)
        pltpu.async_copy(rows_c.at[:, pl.ds(0, HIDDEN)], out_slice(c), semo_c)

    def pair_body(cc, carry):
        c0 = cc * 2
        step(c0, bufs[0], bufs[1])
        step(c0 + 1, bufs[1], bufs[0])
        return carry

    pltpu.sync_copy(x_hbm.at[wid, 0], idx0)
    _fire_gathers(table_hbm, idx0, rows0, semg0)

    lax.fori_loop(0, _NCH // 2, pair_body, 0)

    pltpu.make_async_copy(
        rows0.at[:, pl.ds(0, HIDDEN)], out_slice(_NCH - 2), semo0
    ).wait()
    pltpu.make_async_copy(
        rows1.at[:, pl.ds(0, HIDDEN)], out_slice(_NCH - 1), semo1
    ).wait()


def kernel(x, shared_weights):
    b_total = x.size
    assert b_total == _B

    xr = x.astype(jnp.int32).reshape(_NW, _NCH, _K, 128)

    wpad = jnp.pad(shared_weights, ((0, 0), (0, 128 - HIDDEN)))
    row_ids = lax.broadcasted_iota(jnp.int32, wpad.shape, 0)
    wprep = jnp.where(row_ids == 0, jnp.float32(0.0),
                      wpad * jnp.float32(SCALE))

    mesh = plsc.VectorSubcoreMesh(core_axis_name="c", subcore_axis_name="s")
    run = pl.kernel(
        _body,
        out_type=jax.ShapeDtypeStruct((_B, 128), jnp.float32),
        mesh=mesh,
        scratch_types=[
            pltpu.VMEM((_K, 128), jnp.int32),
            pltpu.VMEM((_K, 128), jnp.int32),
            pltpu.VMEM((_C, 128), jnp.float32),
            pltpu.VMEM((_C, 128), jnp.float32),
            pltpu.SemaphoreType.DMA,
            pltpu.SemaphoreType.DMA,
            pltpu.SemaphoreType.DMA,
            pltpu.SemaphoreType.DMA,
        ],
        compiler_params=pltpu.CompilerParams(use_tc_tiling_on_sc=False),
    )
    out = run(xr, wprep)
    return out.reshape(4096, 200, 128)[:, :, :HIDDEN]
